# Initial kernel scaffold; baseline (speedup 1.0000x reference)
import functools

import jax
import jax.numpy as jnp
from jax import lax
from jax.experimental import pallas as pl
from jax.experimental.pallas import tpu as pltpu
from jax.experimental.pallas import tpu_sc as plsc

NC = 2
NS = 16
L = 16

NW = NC * NS
C = 128
N_NODES = 10000
D = 128
NPAD = 10112
ROWS_PER_SUB = NPAD // NS
N_EDGES = 320000
CH_PER_W = -(-N_EDGES // (NW * C))
TOT_CHUNKS = CH_PER_W * NW
EPAD = TOT_CHUNKS * C

_MESH = plsc.VectorSubcoreMesh(core_axis_name="c", subcore_axis_name="s")


def _agg_body(with_cnt, x_hbm, src_hbm, dst_hbm, *refs):
    if with_cnt:
        (out_agg, out_cnt, src_v, dst_v, rows_v, zero_v,
         ones_v, zero16_v, agg_sh, cnt_sh) = refs
    else:
        out_agg, src_v, dst_v, rows_v, zero_v, agg_sh = refs

    cid = lax.axis_index("c")
    sid = lax.axis_index("s")
    wid = cid * NS + sid
    base = sid * ROWS_PER_SUB

    zf = jnp.zeros((L,), jnp.float32)
    for i in range(8):
        for j in range(D // L):
            zero_v[i, pl.ds(j * L, L)] = zf
    if with_cnt:
        of = jnp.full((L,), 1.0, jnp.float32)
        for i in range(8):
            zero16_v[i, pl.ds(0, L)] = zf

        @pl.loop(0, C)
        def _(i):
            ones_v[i, pl.ds(0, L)] = of

    @pl.loop(0, ROWS_PER_SUB // 8)
    def _(r):
        pltpu.sync_copy(zero_v, agg_sh.at[pl.ds(base + r * 8, 8)])

    if with_cnt:
        @pl.loop(0, ROWS_PER_SUB // 8)
        def _(r):
            pltpu.sync_copy(zero16_v, cnt_sh.at[pl.ds(base + r * 8, 8)])

    plsc.subcore_barrier()

    @pl.loop(0, CH_PER_W)
    def _(i):
        chunk = wid * CH_PER_W + i
        pltpu.sync_copy(src_hbm.at[chunk], src_v.at[0])
        pltpu.sync_copy(dst_hbm.at[chunk], dst_v.at[0])
        pltpu.sync_copy(x_hbm.at[src_v.at[0]], rows_v)
        pltpu.sync_copy(rows_v, agg_sh.at[dst_v.at[0]], add=True)
        if with_cnt:
            pltpu.sync_copy(ones_v, cnt_sh.at[dst_v.at[0]], add=True)

    plsc.subcore_barrier()

    pltpu.sync_copy(agg_sh.at[pl.ds(base, ROWS_PER_SUB)],
                    out_agg.at[cid, pl.ds(base, ROWS_PER_SUB)])
    if with_cnt:
        pltpu.sync_copy(cnt_sh.at[pl.ds(base, ROWS_PER_SUB)],
                        out_cnt.at[cid, pl.ds(base, ROWS_PER_SUB)])


def _make_agg(with_cnt):
    out_type = [jax.ShapeDtypeStruct((NC, NPAD, D), jnp.float32)]
    scratch = [
        pltpu.VMEM((1, C), jnp.int32),
        pltpu.VMEM((1, C), jnp.int32),
        pltpu.VMEM((C, D), jnp.float32),
        pltpu.VMEM((8, D), jnp.float32),
    ]
    if with_cnt:
        out_type.append(jax.ShapeDtypeStruct((NC, NPAD, L), jnp.float32))
        scratch += [
            pltpu.VMEM((C, L), jnp.float32),
            pltpu.VMEM((8, L), jnp.float32),
        ]
    scratch.append(pltpu.VMEM_SHARED((NPAD, D), jnp.float32))
    if with_cnt:
        scratch.append(pltpu.VMEM_SHARED((NPAD, L), jnp.float32))
    return pl.kernel(
        functools.partial(_agg_body, with_cnt),
        out_type=out_type if with_cnt else out_type[0],
        mesh=_MESH,
        scratch_types=scratch,
    )


_agg_with_cnt = _make_agg(True)
_agg_no_cnt = _make_agg(False)


def _tc_body(relu, agg_ref, cnt_ref, x_ref, wl_ref, wr_ref, b_ref, o_ref):
    agg = agg_ref[0] + agg_ref[1]
    cnt = cnt_ref[0, :, 0:1] + cnt_ref[1, :, 0:1]
    mean = agg / jnp.maximum(cnt, 1.0)
    o = (jnp.dot(mean, wl_ref[...], preferred_element_type=jnp.float32,
                 precision=lax.Precision.HIGHEST)
         + b_ref[...]
         + jnp.dot(x_ref[...], wr_ref[...], preferred_element_type=jnp.float32,
                   precision=lax.Precision.HIGHEST))
    o_ref[...] = jnp.maximum(o, 0.0) if relu else o


def _tc_layer(aggp, cntp, x, W_l, b_l, W_r, relu):
    blk = 1000
    return pl.pallas_call(
        functools.partial(_tc_body, relu),
        grid=(N_NODES // blk,),
        in_specs=[
            pl.BlockSpec((NC, blk, D), lambda i: (0, i, 0)),
            pl.BlockSpec((NC, blk, L), lambda i: (0, i, 0)),
            pl.BlockSpec((blk, D), lambda i: (i, 0)),
            pl.BlockSpec((D, D), lambda i: (0, 0)),
            pl.BlockSpec((D, D), lambda i: (0, 0)),
            pl.BlockSpec((1, D), lambda i: (0, 0)),
        ],
        out_specs=pl.BlockSpec((blk, D), lambda i: (i, 0)),
        out_shape=jax.ShapeDtypeStruct((N_NODES, D), jnp.float32),
    )(aggp, cntp, x, W_l.T, W_r.T, b_l.reshape(1, D))


@jax.jit
def kernel(x, edge_index, W_l1, b_l1, W_r1, W_l2, b_l2, W_r2):
    src = edge_index[0].astype(jnp.int32)
    dst = edge_index[1].astype(jnp.int32)
    pad = EPAD - N_EDGES
    src = jnp.concatenate([src, jnp.zeros((pad,), jnp.int32)])
    dst = jnp.concatenate([dst, jnp.full((pad,), N_NODES, jnp.int32)])
    src = src.reshape(TOT_CHUNKS, C)
    dst = dst.reshape(TOT_CHUNKS, C)

    agg1, cnt = _agg_with_cnt(x, src, dst)
    h = _tc_layer(agg1, cnt, x, W_l1, b_l1, W_r1, relu=True)
    agg2 = _agg_no_cnt(h, src, dst)
    return _tc_layer(agg2, cnt, h, W_l2, b_l2, W_r2, relu=False)


# speedup vs baseline: 3.7495x; 3.7495x over previous
import functools

import jax
import jax.numpy as jnp
from jax import lax
from jax.experimental import pallas as pl
from jax.experimental.pallas import tpu as pltpu
from jax.experimental.pallas import tpu_sc as plsc

NC = 2
NS = 16

NW = NC * NS
C = 128
N_NODES = 10000
D = 128
NPAD = 10112
ROWS_PER_SUB = NPAD // NS
NBLK = ROWS_PER_SUB // 8
N_EDGES = 320000
CH_PER_W = -(-N_EDGES // (NW * C))
TOT_CHUNKS = CH_PER_W * NW
EPAD = TOT_CHUNKS * C

_MESH = plsc.VectorSubcoreMesh(core_axis_name="c", subcore_axis_name="s")


def _agg_body(x_hbm, src_hbm, dst_hbm, z8_hbm, *refs):
    out_agg, src_v, dst_v, rows_v, zero_v, stage_v, agg_sh, sem = refs

    cid = lax.axis_index("c")
    sid = lax.axis_index("s")
    wid = cid * NS + sid
    base = sid * ROWS_PER_SUB

    pltpu.sync_copy(z8_hbm, zero_v)

    @pl.loop(0, NBLK)
    def _(r):
        pltpu.sync_copy(zero_v, agg_sh.at[pl.ds(base + r * 8, 8)])

    plsc.subcore_barrier()

    @pl.loop(0, CH_PER_W)
    def _(i):
        chunk = wid * CH_PER_W + i
        pltpu.sync_copy(src_hbm.at[chunk], src_v.at[0])
        pltpu.sync_copy(dst_hbm.at[chunk], dst_v.at[0])
        pltpu.async_copy(x_hbm.at[src_v.at[0]], rows_v, sem).wait()
        pltpu.sync_copy(rows_v, agg_sh.at[dst_v.at[0]], add=True)

    plsc.subcore_barrier()

    @pl.loop(0, NBLK)
    def _(r):
        pltpu.sync_copy(agg_sh.at[pl.ds(base + r * 8, 8)], stage_v)
        pltpu.sync_copy(stage_v, out_agg.at[wid * NBLK + r])


_agg = pl.kernel(
    _agg_body,
    out_type=jax.ShapeDtypeStruct((NW * NBLK, 8, D), jnp.float32),
    mesh=_MESH,
    scratch_types=[
        pltpu.VMEM((1, C), jnp.int32),
        pltpu.VMEM((1, C), jnp.int32),
        pltpu.VMEM((C, D), jnp.float32),
        pltpu.VMEM((8, D), jnp.float32),
        pltpu.VMEM((8, D), jnp.float32),
        pltpu.VMEM_SHARED((NPAD, D), jnp.float32),
        pltpu.SemaphoreType.DMA,
    ],
)


def _cnt_body(dst_hbm, ones_hbm, z8_hbm, *refs):
    out_cnt, dst_v, ones_v, zero_v, stage_v, cnt_sh = refs

    cid = lax.axis_index("c")
    sid = lax.axis_index("s")
    wid = cid * NS + sid
    base = sid * ROWS_PER_SUB

    pltpu.sync_copy(z8_hbm, zero_v)
    pltpu.sync_copy(ones_hbm, ones_v)

    @pl.loop(0, NBLK)
    def _(r):
        pltpu.sync_copy(zero_v, cnt_sh.at[pl.ds(base + r * 8, 8)])

    plsc.subcore_barrier()

    @pl.loop(0, CH_PER_W)
    def _(i):
        chunk = wid * CH_PER_W + i
        pltpu.sync_copy(dst_hbm.at[chunk], dst_v.at[0])
        pltpu.sync_copy(ones_v, cnt_sh.at[dst_v.at[0]], add=True)

    plsc.subcore_barrier()

    @pl.loop(0, NBLK)
    def _(r):
        pltpu.sync_copy(cnt_sh.at[pl.ds(base + r * 8, 8)], stage_v)
        pltpu.sync_copy(stage_v, out_cnt.at[wid * NBLK + r])


_cnt = pl.kernel(
    _cnt_body,
    out_type=jax.ShapeDtypeStruct((NW * NBLK, 8, D), jnp.float32),
    mesh=_MESH,
    scratch_types=[
        pltpu.VMEM((1, C), jnp.int32),
        pltpu.VMEM((C, D), jnp.float32),
        pltpu.VMEM((8, D), jnp.float32),
        pltpu.VMEM((8, D), jnp.float32),
        pltpu.VMEM_SHARED((NPAD, D), jnp.float32),
    ],
)


def _tc_body(relu, agg_ref, cnt_ref, x_ref, wl_ref, wr_ref, b_ref, o_ref):
    agg = agg_ref[0] + agg_ref[1]
    cnt = cnt_ref[0] + cnt_ref[1]
    mean = agg / jnp.maximum(cnt, 1.0)
    o = (jnp.dot(mean, wl_ref[...], preferred_element_type=jnp.float32,
                 precision=lax.Precision.HIGHEST)
         + b_ref[...]
         + jnp.dot(x_ref[...], wr_ref[...], preferred_element_type=jnp.float32,
                   precision=lax.Precision.HIGHEST))
    o_ref[...] = jnp.maximum(o, 0.0) if relu else o


def _tc_layer(aggp, cntp, x, W_l, b_l, W_r, relu):
    blk = 1000
    return pl.pallas_call(
        functools.partial(_tc_body, relu),
        grid=(N_NODES // blk,),
        in_specs=[
            pl.BlockSpec((NC, blk, D), lambda i: (0, i, 0)),
            pl.BlockSpec((NC, blk, 1), lambda i: (0, i, 0)),
            pl.BlockSpec((blk, D), lambda i: (i, 0)),
            pl.BlockSpec((D, D), lambda i: (0, 0)),
            pl.BlockSpec((D, D), lambda i: (0, 0)),
            pl.BlockSpec((1, D), lambda i: (0, 0)),
        ],
        out_specs=pl.BlockSpec((blk, D), lambda i: (i, 0)),
        out_shape=jax.ShapeDtypeStruct((N_NODES, D), jnp.float32),
    )(aggp, cntp, x, W_l.T, W_r.T, b_l.reshape(1, D))


@jax.jit
def kernel(x, edge_index, W_l1, b_l1, W_r1, W_l2, b_l2, W_r2):
    src = edge_index[0].astype(jnp.int32)
    dst = edge_index[1].astype(jnp.int32)
    pad = EPAD - N_EDGES
    src = jnp.concatenate([src, jnp.zeros((pad,), jnp.int32)])
    dst = jnp.concatenate([dst, jnp.full((pad,), N_NODES, jnp.int32)])
    src = src.reshape(TOT_CHUNKS, C)
    dst = dst.reshape(TOT_CHUNKS, C)

    z8 = jnp.zeros((8, D), jnp.float32)
    ones = jnp.ones((C, D), jnp.float32)

    cntp = _cnt(dst, ones, z8).reshape(NC, NPAD, D)[:, :, 0:1]
    agg1 = _agg(x, src, dst, z8).reshape(NC, NPAD, D)
    h = _tc_layer(agg1, cntp, x, W_l1, b_l1, W_r1, relu=True)
    agg2 = _agg(h, src, dst, z8).reshape(NC, NPAD, D)
    return _tc_layer(agg2, cntp, h, W_l2, b_l2, W_r2, relu=False)
